# baseline (speedup 1.0000x reference)
import jax
import jax.numpy as jnp
from jax import lax
from jax.experimental import pallas as pl
from jax.experimental.pallas import tpu as pltpu, tpu_sc as plsc

_NW = 32
_BANDS = 8
_ROWS = 64
_W = 512
_CSLABS = 80 * _BANDS // _NW
_MSLABS = 16 * _BANDS // _NW
_NB = 3


def _sigmoid_inplace(buf):
    def row(r, carry):
        def col(j, carry2):
            sl = pl.ds(j * 16, 16)
            v = buf[r, sl]
            buf[r, sl] = 1.0 / (1.0 + jnp.exp(-v))
            return carry2

        return lax.fori_loop(0, _W // 16, col, carry, unroll=8)

    lax.fori_loop(0, _ROWS, row, 0)


def _sc_body(x_hbm, m_hbm, o_hbm, b0, b1, b2, i0, i1, i2, o0, o1, o2):
    bufs = (b0, b1, b2)
    sin = (i0, i1, i2)
    sout = (o0, o1, o2)
    c = lax.axis_index("c")
    s = lax.axis_index("s")
    wid = s * 2 + c

    jobs = []
    for k in range(_CSLABS):
        idx = wid * _CSLABS + k
        ch = idx // _BANDS
        band = idx % _BANDS
        jobs.append((x_hbm, ch, ch, band, False))
    for k in range(_MSLABS):
        idx = wid * _MSLABS + k
        ch = idx // _BANDS
        band = idx % _BANDS
        jobs.append((m_hbm, ch, ch + 80, band, True))

    def in_cp(job, b):
        src, sch, _, band, _ = job
        return pltpu.make_async_copy(
            src.at[sch, pl.ds(band * _ROWS, _ROWS), :], bufs[b], sin[b]
        )

    def out_cp(job, b):
        _, _, dch, band, _ = job
        return pltpu.make_async_copy(
            bufs[b], o_hbm.at[dch, pl.ds(band * _ROWS, _ROWS), :], sout[b]
        )

    n = len(jobs)
    for b in range(_NB):
        in_cp(jobs[b], b).start()
    for k in range(n):
        b = k % _NB
        in_cp(jobs[k], b).wait()
        if jobs[k][4]:
            _sigmoid_inplace(bufs[b])
        out_cp(jobs[k], b).start()
        if k + _NB < n:
            out_cp(jobs[k], b).wait()
            in_cp(jobs[k + _NB], b).start()
    for k in range(max(0, n - _NB), n):
        out_cp(jobs[k], k % _NB).wait()


def kernel(x, mask):
    mesh = plsc.VectorSubcoreMesh(core_axis_name="c", subcore_axis_name="s")
    return pl.kernel(
        _sc_body,
        mesh=mesh,
        out_type=jax.ShapeDtypeStruct(x.shape, x.dtype),
        compiler_params=pltpu.CompilerParams(use_tc_tiling_on_sc=True),
        scratch_types=[
            pltpu.VMEM((_ROWS, _W), jnp.float32),
            pltpu.VMEM((_ROWS, _W), jnp.float32),
            pltpu.VMEM((_ROWS, _W), jnp.float32),
            pltpu.SemaphoreType.DMA,
            pltpu.SemaphoreType.DMA,
            pltpu.SemaphoreType.DMA,
            pltpu.SemaphoreType.DMA,
            pltpu.SemaphoreType.DMA,
            pltpu.SemaphoreType.DMA,
        ],
    )(x, mask)


# speedup vs baseline: 1.0146x; 1.0146x over previous
import jax
import jax.numpy as jnp
from jax import lax
from jax.experimental import pallas as pl
from jax.experimental.pallas import tpu as pltpu, tpu_sc as plsc

_NW = 32
_BANDS = 8
_ROWS = 64
_W = 512
_CSLABS = 80 * _BANDS // _NW
_MSLABS = 16 * _BANDS // _NW
_NB = 3


def _sigmoid_inplace(buf):
    def row(r, carry):
        def col(j, carry2):
            sl = pl.ds(j * 16, 16)
            v = buf[r, sl]
            buf[r, sl] = 1.0 / (1.0 + jnp.exp(-v))
            return carry2

        return lax.fori_loop(0, _W // 16, col, carry, unroll=8)

    lax.fori_loop(0, _ROWS, row, 0)


def _sc_body(x_hbm, m_hbm, o_hbm, b0, b1, b2, i0, i1, i2, o0, o1, o2):
    bufs = (b0, b1, b2)
    sin = (i0, i1, i2)
    sout = (o0, o1, o2)
    c = lax.axis_index("c")
    s = lax.axis_index("s")
    wid = s * 2 + c

    cjobs = []
    for k in range(_CSLABS):
        idx = wid * _CSLABS + k
        ch = idx // _BANDS
        band = idx % _BANDS
        cjobs.append((x_hbm, ch, ch, band, False))
    mjobs = []
    for k in range(_MSLABS):
        idx = wid * _MSLABS + k
        ch = idx // _BANDS
        band = idx % _BANDS
        mjobs.append((m_hbm, ch, ch + 80, band, True))
    jobs = []
    for k in range(_MSLABS):
        jobs.extend(cjobs[k * 5:(k + 1) * 5])
        jobs.append(mjobs[k])

    def in_cp(job, b):
        src, sch, _, band, _ = job
        return pltpu.make_async_copy(
            src.at[sch, pl.ds(band * _ROWS, _ROWS), :], bufs[b], sin[b]
        )

    def out_cp(job, b):
        _, _, dch, band, _ = job
        return pltpu.make_async_copy(
            bufs[b], o_hbm.at[dch, pl.ds(band * _ROWS, _ROWS), :], sout[b]
        )

    n = len(jobs)
    in_cp(jobs[0], 0).start()
    in_cp(jobs[1], 1).start()
    for k in range(n):
        b = k % _NB
        in_cp(jobs[k], b).wait()
        if jobs[k][4]:
            _sigmoid_inplace(bufs[b])
        out_cp(jobs[k], b).start()
        if k + 2 < n:
            b2 = (k + 2) % _NB
            if k >= 1:
                out_cp(jobs[k - 1], b2).wait()
            in_cp(jobs[k + 2], b2).start()
    for k in (n - 3, n - 2, n - 1):
        out_cp(jobs[k], k % _NB).wait()


def kernel(x, mask):
    mesh = plsc.VectorSubcoreMesh(core_axis_name="c", subcore_axis_name="s")
    return pl.kernel(
        _sc_body,
        mesh=mesh,
        out_type=jax.ShapeDtypeStruct(x.shape, x.dtype),
        compiler_params=pltpu.CompilerParams(use_tc_tiling_on_sc=True),
        scratch_types=[
            pltpu.VMEM((_ROWS, _W), jnp.float32),
            pltpu.VMEM((_ROWS, _W), jnp.float32),
            pltpu.VMEM((_ROWS, _W), jnp.float32),
            pltpu.SemaphoreType.DMA,
            pltpu.SemaphoreType.DMA,
            pltpu.SemaphoreType.DMA,
            pltpu.SemaphoreType.DMA,
            pltpu.SemaphoreType.DMA,
            pltpu.SemaphoreType.DMA,
        ],
    )(x, mask)


# speedup vs baseline: 1.0677x; 1.0524x over previous
import jax
import jax.numpy as jnp
from jax import lax
from jax.experimental import pallas as pl
from jax.experimental.pallas import tpu as pltpu, tpu_sc as plsc

_NW = 32
_BANDS = 8
_ROWS = 64
_W = 512
_MSLABS = 16 * _BANDS // _NW
_NB = 3
_CB = 8
_NCOPY = 80 // _CB


def _sigmoid_inplace(buf):
    def row(r, carry):
        def col(j, carry2):
            sl = pl.ds(j * 16, 16)
            v = buf[r, sl]
            buf[r, sl] = 1.0 / (1.0 + jnp.exp(-v))
            return carry2

        return lax.fori_loop(0, _W // 16, col, carry, unroll=8)

    lax.fori_loop(0, _ROWS, row, 0)


def _sc_body(m_hbm, o_hbm, b0, b1, b2, i0, i1, i2, o0, o1, o2):
    bufs = (b0, b1, b2)
    sin = (i0, i1, i2)
    sout = (o0, o1, o2)
    c = lax.axis_index("c")
    s = lax.axis_index("s")
    wid = s * 2 + c

    jobs = []
    for k in range(_MSLABS):
        idx = wid * _MSLABS + k
        ch = idx // _BANDS
        band = idx % _BANDS
        jobs.append((ch, band))

    def in_cp(job, b):
        ch, band = job
        return pltpu.make_async_copy(
            m_hbm.at[ch, pl.ds(band * _ROWS, _ROWS), :], bufs[b], sin[b]
        )

    def out_cp(job, b):
        ch, band = job
        return pltpu.make_async_copy(
            bufs[b], o_hbm.at[ch + 80, pl.ds(band * _ROWS, _ROWS), :], sout[b]
        )

    n = len(jobs)
    in_cp(jobs[0], 0).start()
    in_cp(jobs[1], 1).start()
    for k in range(n):
        b = k % _NB
        in_cp(jobs[k], b).wait()
        _sigmoid_inplace(bufs[b])
        out_cp(jobs[k], b).start()
        if k + 2 < n:
            b2 = (k + 2) % _NB
            if k >= 1:
                out_cp(jobs[k - 1], b2).wait()
            in_cp(jobs[k + 2], b2).start()
    for k in (n - 3, n - 2, n - 1):
        if k >= 0:
            out_cp(jobs[k], k % _NB).wait()


def _tc_copy(x_ref, o1_ref, out_ref):
    out_ref[...] = x_ref[...]


def kernel(x, mask):
    C, H, W = x.shape
    mesh = plsc.VectorSubcoreMesh(core_axis_name="c", subcore_axis_name="s")
    o1 = pl.kernel(
        _sc_body,
        mesh=mesh,
        out_type=jax.ShapeDtypeStruct(x.shape, x.dtype),
        compiler_params=pltpu.CompilerParams(use_tc_tiling_on_sc=True),
        scratch_types=[
            pltpu.VMEM((_ROWS, _W), jnp.float32),
            pltpu.VMEM((_ROWS, _W), jnp.float32),
            pltpu.VMEM((_ROWS, _W), jnp.float32),
            pltpu.SemaphoreType.DMA,
            pltpu.SemaphoreType.DMA,
            pltpu.SemaphoreType.DMA,
            pltpu.SemaphoreType.DMA,
            pltpu.SemaphoreType.DMA,
            pltpu.SemaphoreType.DMA,
        ],
    )(mask)
    return pl.pallas_call(
        _tc_copy,
        grid=(_NCOPY,),
        in_specs=[
            pl.BlockSpec((_CB, H, W), lambda c: (c, 0, 0)),
            pl.BlockSpec(memory_space=pltpu.MemorySpace.HBM),
        ],
        out_specs=pl.BlockSpec((_CB, H, W), lambda c: (c, 0, 0)),
        out_shape=jax.ShapeDtypeStruct((C, H, W), x.dtype),
        input_output_aliases={1: 0},
    )(x, o1)


# speedup vs baseline: 1.1782x; 1.1034x over previous
import jax
import jax.numpy as jnp
from jax import lax
from jax.experimental import pallas as pl
from jax.experimental.pallas import tpu as pltpu, tpu_sc as plsc

_NW = 32
_BANDS = 8
_ROWS = 64
_W = 512
_MSLABS = 16 * _BANDS // _NW
_NB = 3
_CB = 8
_NCOPY = 80 // _CB


def _sigmoid_inplace(buf):
    def row(r, carry):
        for j in range(_W // 16):
            sl = pl.ds(j * 16, 16)
            v = buf[r, sl]
            buf[r, sl] = 1.0 / (1.0 + jnp.exp(-v))
        return carry

    lax.fori_loop(0, _ROWS, row, 0)


def _sc_body(m_hbm, o_hbm, b0, b1, b2, i0, i1, i2, o0, o1, o2):
    bufs = (b0, b1, b2)
    sin = (i0, i1, i2)
    sout = (o0, o1, o2)
    c = lax.axis_index("c")
    s = lax.axis_index("s")
    wid = s * 2 + c

    jobs = []
    for k in range(_MSLABS):
        idx = wid * _MSLABS + k
        ch = idx // _BANDS
        band = idx % _BANDS
        jobs.append((ch, band))

    def in_cp(job, b):
        ch, band = job
        return pltpu.make_async_copy(
            m_hbm.at[ch, pl.ds(band * _ROWS, _ROWS), :], bufs[b], sin[b]
        )

    def out_cp(job, b):
        ch, band = job
        return pltpu.make_async_copy(
            bufs[b], o_hbm.at[ch + 80, pl.ds(band * _ROWS, _ROWS), :], sout[b]
        )

    n = len(jobs)
    in_cp(jobs[0], 0).start()
    in_cp(jobs[1], 1).start()
    for k in range(n):
        b = k % _NB
        in_cp(jobs[k], b).wait()
        _sigmoid_inplace(bufs[b])
        out_cp(jobs[k], b).start()
        if k + 2 < n:
            b2 = (k + 2) % _NB
            if k >= 1:
                out_cp(jobs[k - 1], b2).wait()
            in_cp(jobs[k + 2], b2).start()
    for k in (n - 3, n - 2, n - 1):
        if k >= 0:
            out_cp(jobs[k], k % _NB).wait()


def _tc_copy(x_ref, o1_ref, out_ref):
    out_ref[...] = x_ref[...]


def kernel(x, mask):
    C, H, W = x.shape
    mesh = plsc.VectorSubcoreMesh(core_axis_name="c", subcore_axis_name="s")
    o1 = pl.kernel(
        _sc_body,
        mesh=mesh,
        out_type=jax.ShapeDtypeStruct(x.shape, x.dtype),
        compiler_params=pltpu.CompilerParams(use_tc_tiling_on_sc=True),
        scratch_types=[
            pltpu.VMEM((_ROWS, _W), jnp.float32),
            pltpu.VMEM((_ROWS, _W), jnp.float32),
            pltpu.VMEM((_ROWS, _W), jnp.float32),
            pltpu.SemaphoreType.DMA,
            pltpu.SemaphoreType.DMA,
            pltpu.SemaphoreType.DMA,
            pltpu.SemaphoreType.DMA,
            pltpu.SemaphoreType.DMA,
            pltpu.SemaphoreType.DMA,
        ],
    )(mask)
    return pl.pallas_call(
        _tc_copy,
        grid=(_NCOPY,),
        in_specs=[
            pl.BlockSpec((_CB, H, W), lambda c: (c, 0, 0)),
            pl.BlockSpec(memory_space=pltpu.MemorySpace.HBM),
        ],
        out_specs=pl.BlockSpec((_CB, H, W), lambda c: (c, 0, 0)),
        out_shape=jax.ShapeDtypeStruct((C, H, W), x.dtype),
        input_output_aliases={1: 0},
    )(x, o1)
